# Initial kernel scaffold; baseline (speedup 1.0000x reference)
import jax
import jax.numpy as jnp
from jax.experimental import pallas as pl


def kernel(X, edge_index, W1, att_src1, att_dst1, b1, W2, att_src2, att_dst2, b2, Wfc, bfc):
    raise NotImplementedError("write your pallas kernel here")


# speedup vs baseline: 1.1233x; 1.1233x over previous
import functools

import jax
import jax.numpy as jnp
import numpy as np
from jax.experimental import pallas as pl
from jax.experimental.pallas import tpu as pltpu

N_NODES = 10000
N_EDGES = 160000
D_IN = 256
HID = 256
HEADS = 8
D_OUT = 128

ROW_BLK = 400
_EPS = 1e-16


def _tc1_body(x_ref, w1_ref, acat_ref, xw_ref, a_ref, gmax_ref):
    i = pl.program_id(0)
    xw = jnp.dot(x_ref[...], w1_ref[...], preferred_element_type=jnp.float32)
    xw_ref[...] = xw
    a = jnp.dot(xw, acat_ref[...], preferred_element_type=jnp.float32)
    a_ref[...] = a
    blkmax = jnp.max(a, axis=0, keepdims=True)

    @pl.when(i == 0)
    def _():
        gmax_ref[...] = blkmax

    @pl.when(i != 0)
    def _():
        gmax_ref[...] = jnp.maximum(gmax_ref[...], blkmax)


def _tc1(X, W1, Acat):
    grid = N_NODES // ROW_BLK
    return pl.pallas_call(
        _tc1_body,
        grid=(grid,),
        in_specs=[
            pl.BlockSpec((ROW_BLK, D_IN), lambda i: (i, 0)),
            pl.BlockSpec((D_IN, HEADS * HID), lambda i: (0, 0)),
            pl.BlockSpec((HEADS * HID, 16), lambda i: (0, 0)),
        ],
        out_specs=[
            pl.BlockSpec((ROW_BLK, HEADS * HID), lambda i: (i, 0)),
            pl.BlockSpec((ROW_BLK, 16), lambda i: (i, 0)),
            pl.BlockSpec((1, 16), lambda i: (0, 0)),
        ],
        out_shape=[
            jax.ShapeDtypeStruct((N_NODES, HEADS * HID), jnp.float32),
            jax.ShapeDtypeStruct((N_NODES, 16), jnp.float32),
            jax.ShapeDtypeStruct((1, 16), jnp.float32),
        ],
    )(X, W1, Acat)


def _tc2_body(agg_ref, dp_ref, k1_ref, e8_ref, b1_ref, w2_ref, a2c_ref,
              xw2_ref, a2_ref, gmax2_ref):
    i = pl.program_id(0)
    denom = jnp.dot(dp_ref[...], k1_ref[...], preferred_element_type=jnp.float32)
    recip = 1.0 / (denom + _EPS)
    scale = jnp.dot(recip, e8_ref[...], preferred_element_type=jnp.float32)
    h = jnp.maximum(agg_ref[...] * scale + b1_ref[...], 0.0)
    xw2 = jnp.dot(h, w2_ref[...], preferred_element_type=jnp.float32)
    xw2_ref[...] = xw2
    a2 = jnp.dot(xw2, a2c_ref[...], preferred_element_type=jnp.float32)
    a2_ref[...] = a2
    blkmax = jnp.max(a2, axis=0, keepdims=True)

    @pl.when(i == 0)
    def _():
        gmax2_ref[...] = blkmax

    @pl.when(i != 0)
    def _():
        gmax2_ref[...] = jnp.maximum(gmax2_ref[...], blkmax)


def _tc2(agg1, dp1T, K1, E8, b1, W2, att2cat):
    grid = N_NODES // ROW_BLK
    return pl.pallas_call(
        _tc2_body,
        grid=(grid,),
        in_specs=[
            pl.BlockSpec((ROW_BLK, HEADS * HID), lambda i: (i, 0)),
            pl.BlockSpec((ROW_BLK, 32), lambda i: (i, 0)),
            pl.BlockSpec((32, HEADS), lambda i: (0, 0)),
            pl.BlockSpec((HEADS, HEADS * HID), lambda i: (0, 0)),
            pl.BlockSpec((1, HEADS * HID), lambda i: (0, 0)),
            pl.BlockSpec((HEADS * HID, HID), lambda i: (0, 0)),
            pl.BlockSpec((HID, 16), lambda i: (0, 0)),
        ],
        out_specs=[
            pl.BlockSpec((ROW_BLK, HID), lambda i: (i, 0)),
            pl.BlockSpec((ROW_BLK, 16), lambda i: (i, 0)),
            pl.BlockSpec((1, 16), lambda i: (0, 0)),
        ],
        out_shape=[
            jax.ShapeDtypeStruct((N_NODES, HID), jnp.float32),
            jax.ShapeDtypeStruct((N_NODES, 16), jnp.float32),
            jax.ShapeDtypeStruct((1, 16), jnp.float32),
        ],
    )(agg1, dp1T, K1, E8, b1, W2, att2cat)


def _tc3_body(agg_ref, dp_ref, ones_ref, e1_ref, b2_ref, wfc_ref, bfc_ref,
              out_ref):
    denom = jnp.dot(dp_ref[...], ones_ref[...], preferred_element_type=jnp.float32)
    recip = 1.0 / (denom + _EPS)
    scale = jnp.dot(recip, e1_ref[...], preferred_element_type=jnp.float32)
    h = jnp.maximum(agg_ref[...] * scale + b2_ref[...], 0.0)
    out_ref[...] = jnp.dot(h, wfc_ref[...], preferred_element_type=jnp.float32) + bfc_ref[...]


def _tc3(agg2, dp2T, ones32, E1, b2, Wfc, bfc):
    grid = N_NODES // ROW_BLK
    return pl.pallas_call(
        _tc3_body,
        grid=(grid,),
        in_specs=[
            pl.BlockSpec((ROW_BLK, HID), lambda i: (i, 0)),
            pl.BlockSpec((ROW_BLK, 32), lambda i: (i, 0)),
            pl.BlockSpec((32, 8), lambda i: (0, 0)),
            pl.BlockSpec((8, HID), lambda i: (0, 0)),
            pl.BlockSpec((1, HID), lambda i: (0, 0)),
            pl.BlockSpec((HID, D_OUT), lambda i: (0, 0)),
            pl.BlockSpec((1, D_OUT), lambda i: (0, 0)),
        ],
        out_specs=[pl.BlockSpec((ROW_BLK, D_OUT), lambda i: (i, 0))],
        out_shape=[jax.ShapeDtypeStruct((N_NODES, D_OUT), jnp.float32)],
    )(agg2, dp2T, ones32, E1, b2, Wfc, bfc)


def _edge_phase_jax(a_cat, gvec, src, dst, heads):
    a_src = a_cat[:, :heads]
    a_dst = a_cat[:, 8:8 + heads]
    alpha = a_src[src] + a_dst[dst]
    alpha = jax.nn.leaky_relu(alpha, 0.2)
    ex = jnp.exp(alpha - gvec[None, :heads])
    denom = jax.ops.segment_sum(ex, dst, num_segments=N_NODES)
    return ex, denom


def _agg_jax(ex, xw, src, dst, heads, ch):
    msg = xw.reshape(N_NODES, heads, ch)[src] * ex[:, :, None]
    agg = jax.ops.segment_sum(msg, dst, num_segments=N_NODES)
    return agg.reshape(N_NODES, heads * ch)


def _make_consts():
    e8 = np.repeat(np.eye(HEADS, dtype=np.float32), HID, axis=1)
    k1 = np.tile(np.eye(HEADS, dtype=np.float32), (4, 1))
    ones32 = np.zeros((32, 8), np.float32)
    ones32[:, 0] = 1.0
    e1 = np.zeros((8, HID), np.float32)
    e1[0, :] = 1.0
    return jnp.asarray(e8), jnp.asarray(k1), jnp.asarray(ones32), jnp.asarray(e1)


def kernel(X, edge_index, W1, att_src1, att_dst1, b1, W2, att_src2, att_dst2,
           b2, Wfc, bfc):
    E8, K1, ONES32, E1 = _make_consts()
    src = edge_index[0]
    dst = edge_index[1]

    hh = jnp.arange(HEADS * HID) // HID
    blockdiag = (hh[:, None] == jnp.arange(HEADS)[None, :]).astype(jnp.float32)
    acat = jnp.concatenate(
        [blockdiag * att_src1.reshape(-1)[:, None],
         blockdiag * att_dst1.reshape(-1)[:, None]], axis=1)

    att2cat = jnp.zeros((HID, 16), jnp.float32)
    att2cat = att2cat.at[:, 0].set(att_src2[0]).at[:, 8].set(att_dst2[0])

    xw1, a1cat, gmax1 = _tc1(X, W1, acat)
    g1 = gmax1[0, :HEADS] + gmax1[0, 8:16]

    ex1, denom1 = _edge_phase_jax(a1cat, g1, src, dst, HEADS)
    agg1 = _agg_jax(ex1, xw1, src, dst, HEADS, HID)
    dp1T = jnp.concatenate([denom1, jnp.zeros((N_NODES, 24), jnp.float32)], axis=1)

    xw2, a2cat, gmax2 = _tc2(agg1, dp1T, K1, E8, b1.reshape(1, -1), W2, att2cat)
    g2 = gmax2[0, :1] + gmax2[0, 8:9]

    ex2, denom2 = _edge_phase_jax(a2cat, g2, src, dst, 1)
    agg2 = _agg_jax(ex2, xw2, src, dst, 1, HID)
    dp2T = jnp.concatenate([denom2, jnp.zeros((N_NODES, 31), jnp.float32)], axis=1)

    (out,) = _tc3(agg2, dp2T, ONES32, E1, b2.reshape(1, -1), Wfc,
                  bfc.reshape(1, -1))
    return out


# speedup vs baseline: 2.6343x; 2.3452x over previous
import functools

import jax
import jax.numpy as jnp
import numpy as np
from jax import lax
from jax.experimental import pallas as pl
from jax.experimental.pallas import tpu as pltpu
from jax.experimental.pallas import tpu_sc as plsc

N_NODES = 10000
N_EDGES = 160000
D_IN = 256
HID = 256
HEADS = 8
D_OUT = 128

ROW_BLK = 400
_EPS = 1e-16

_NC = 2
_NS = 16
_L = 16
_NW = _NC * _NS
E_PAD = 163840
_ECH = 5120
N_PAD = 10112


def _tc1_body(x_ref, w1_ref, acat_ref, xw_ref, a_ref, gmax_ref):
    i = pl.program_id(0)
    xw = jnp.dot(x_ref[...], w1_ref[...], preferred_element_type=jnp.float32)
    xw_ref[...] = xw
    a = jnp.dot(xw, acat_ref[...], preferred_element_type=jnp.float32)
    a_ref[...] = a
    blkmax = jnp.max(a, axis=0, keepdims=True)

    @pl.when(i == 0)
    def _():
        gmax_ref[...] = blkmax

    @pl.when(i != 0)
    def _():
        gmax_ref[...] = jnp.maximum(gmax_ref[...], blkmax)


def _tc1(X, W1, Acat):
    grid = N_NODES // ROW_BLK
    return pl.pallas_call(
        _tc1_body,
        grid=(grid,),
        in_specs=[
            pl.BlockSpec((ROW_BLK, D_IN), lambda i: (i, 0)),
            pl.BlockSpec((D_IN, HEADS * HID), lambda i: (0, 0)),
            pl.BlockSpec((HEADS * HID, 16), lambda i: (0, 0)),
        ],
        out_specs=[
            pl.BlockSpec((ROW_BLK, HEADS * HID), lambda i: (i, 0)),
            pl.BlockSpec((ROW_BLK, 16), lambda i: (i, 0)),
            pl.BlockSpec((1, 16), lambda i: (0, 0)),
        ],
        out_shape=[
            jax.ShapeDtypeStruct((N_NODES, HEADS * HID), jnp.float32),
            jax.ShapeDtypeStruct((N_NODES, 16), jnp.float32),
            jax.ShapeDtypeStruct((1, 16), jnp.float32),
        ],
    )(X, W1, Acat)


def _tc2_body(agg_ref, dp_ref, k1_ref, e8_ref, b1_ref, w2_ref, a2c_ref,
              xw2_ref, a2_ref, gmax2_ref):
    i = pl.program_id(0)
    denom = jnp.dot(dp_ref[...], k1_ref[...], preferred_element_type=jnp.float32)
    recip = 1.0 / (denom + _EPS)
    scale = jnp.dot(recip, e8_ref[...], preferred_element_type=jnp.float32)
    h = jnp.maximum(agg_ref[...] * scale + b1_ref[...], 0.0)
    xw2 = jnp.dot(h, w2_ref[...], preferred_element_type=jnp.float32)
    xw2_ref[...] = xw2
    a2 = jnp.dot(xw2, a2c_ref[...], preferred_element_type=jnp.float32)
    a2_ref[...] = a2
    blkmax = jnp.max(a2, axis=0, keepdims=True)

    @pl.when(i == 0)
    def _():
        gmax2_ref[...] = blkmax

    @pl.when(i != 0)
    def _():
        gmax2_ref[...] = jnp.maximum(gmax2_ref[...], blkmax)


def _tc2(agg1, dp1T, K1, E8, b1, W2, att2cat):
    grid = N_NODES // ROW_BLK
    return pl.pallas_call(
        _tc2_body,
        grid=(grid,),
        in_specs=[
            pl.BlockSpec((ROW_BLK, HEADS * HID), lambda i: (i, 0)),
            pl.BlockSpec((ROW_BLK, 32), lambda i: (i, 0)),
            pl.BlockSpec((32, HEADS), lambda i: (0, 0)),
            pl.BlockSpec((HEADS, HEADS * HID), lambda i: (0, 0)),
            pl.BlockSpec((1, HEADS * HID), lambda i: (0, 0)),
            pl.BlockSpec((HEADS * HID, HID), lambda i: (0, 0)),
            pl.BlockSpec((HID, 16), lambda i: (0, 0)),
        ],
        out_specs=[
            pl.BlockSpec((ROW_BLK, HID), lambda i: (i, 0)),
            pl.BlockSpec((ROW_BLK, 16), lambda i: (i, 0)),
            pl.BlockSpec((1, 16), lambda i: (0, 0)),
        ],
        out_shape=[
            jax.ShapeDtypeStruct((N_NODES, HID), jnp.float32),
            jax.ShapeDtypeStruct((N_NODES, 16), jnp.float32),
            jax.ShapeDtypeStruct((1, 16), jnp.float32),
        ],
    )(agg1, dp1T, K1, E8, b1, W2, att2cat)


def _tc3_body(agg_ref, dp_ref, ones_ref, e1_ref, b2_ref, wfc_ref, bfc_ref,
              out_ref):
    denom = jnp.dot(dp_ref[...], ones_ref[...], preferred_element_type=jnp.float32)
    recip = 1.0 / (denom + _EPS)
    scale = jnp.dot(recip, e1_ref[...], preferred_element_type=jnp.float32)
    h = jnp.maximum(agg_ref[...] * scale + b2_ref[...], 0.0)
    out_ref[...] = jnp.dot(h, wfc_ref[...], preferred_element_type=jnp.float32) + bfc_ref[...]


def _tc3(agg2, dp2T, ones32, E1, b2, Wfc, bfc):
    grid = N_NODES // ROW_BLK
    return pl.pallas_call(
        _tc3_body,
        grid=(grid,),
        in_specs=[
            pl.BlockSpec((ROW_BLK, HID), lambda i: (i, 0)),
            pl.BlockSpec((ROW_BLK, 32), lambda i: (i, 0)),
            pl.BlockSpec((32, 8), lambda i: (0, 0)),
            pl.BlockSpec((8, HID), lambda i: (0, 0)),
            pl.BlockSpec((1, HID), lambda i: (0, 0)),
            pl.BlockSpec((HID, D_OUT), lambda i: (0, 0)),
            pl.BlockSpec((1, D_OUT), lambda i: (0, 0)),
        ],
        out_specs=[pl.BlockSpec((ROW_BLK, D_OUT), lambda i: (i, 0))],
        out_shape=[jax.ShapeDtypeStruct((N_NODES, D_OUT), jnp.float32)],
    )(agg2, dp2T, ones32, E1, b2, Wfc, bfc)


@functools.lru_cache(maxsize=None)
def _make_edge_softmax(H):
    nseg = _NW // H
    eper = E_PAD // nseg
    nchunks = eper // _ECH
    mesh = plsc.VectorSubcoreMesh(core_axis_name="c", subcore_axis_name="s",
                                  num_cores=_NC, num_subcores=_NS)

    @functools.partial(
        pl.kernel,
        out_type=[
            jax.ShapeDtypeStruct((H, 1, E_PAD), jnp.float32),
            jax.ShapeDtypeStruct((_NW, 1, N_PAD), jnp.float32),
        ],
        mesh=mesh,
        compiler_params=pltpu.CompilerParams(needs_layout_passes=False),
        scratch_types=[
            pltpu.VMEM((N_PAD,), jnp.float32),
            pltpu.VMEM((N_PAD,), jnp.float32),
            pltpu.VMEM((N_PAD,), jnp.float32),
            pltpu.VMEM((16,), jnp.float32),
            pltpu.VMEM((_ECH,), jnp.int32),
            pltpu.VMEM((_ECH,), jnp.int32),
            pltpu.VMEM((_ECH,), jnp.float32),
        ],
    )
    def edge_softmax(asT, srcp, dstp, g, exT, dpart, asrc_v, adst_v, den_v,
                     g_v, srcv, dstv, exv):
        c = lax.axis_index("c")
        s = lax.axis_index("s")
        wid = s * _NC + c
        if H == 1:
            head = 0
            seg = wid
        else:
            head = wid % H
            seg = wid // H
        pltpu.sync_copy(asT.at[head, 0], asrc_v)
        pltpu.sync_copy(asT.at[8 + head, 0], adst_v)
        pltpu.sync_copy(g, g_v)
        iota = lax.iota(jnp.int32, _L)
        gvv = g_v[...]
        ghead = jnp.max(jnp.where(iota == head, gvv, -3.4e38))
        gb = jnp.full((_L,), ghead)

        def zero_body(i, _):
            den_v[pl.ds(i * _L, _L)] = jnp.zeros((_L,), jnp.float32)
            return 0

        lax.fori_loop(0, N_PAD // _L, zero_body, 0)

        for ci in range(nchunks):
            base = pl.multiple_of(seg * eper + ci * _ECH, _ECH)
            pltpu.sync_copy(srcp.at[pl.ds(base, _ECH)], srcv)
            pltpu.sync_copy(dstp.at[pl.ds(base, _ECH)], dstv)

            def grp(j, _):
                sv = srcv[pl.ds(j * _L, _L)]
                dv = dstv[pl.ds(j * _L, _L)]
                t = plsc.load_gather(asrc_v, [sv]) + plsc.load_gather(adst_v, [dv])
                alpha = jnp.where(t >= 0.0, t, 0.2 * t)
                ex = jnp.exp(alpha - gb)
                eid = base + j * _L + iota
                ex = jnp.where(eid < N_EDGES, ex, 0.0)
                exv[pl.ds(j * _L, _L)] = ex
                plsc.addupdate_scatter(den_v, [dv], ex)
                return 0

            lax.fori_loop(0, _ECH // _L, grp, 0)
            pltpu.sync_copy(exv, exT.at[head, 0, pl.ds(base, _ECH)])
        pltpu.sync_copy(den_v, dpart.at[wid, 0])

    return edge_softmax


@functools.lru_cache(maxsize=None)
def _make_agg(NSLICE, D, Cw, NPASS):
    Dh = D // NSLICE
    NWIN = _NW // NSLICE
    NOUT = NWIN * Cw * NPASS
    ECH2 = 2560
    NCH = E_PAD // ECH2
    nq = Dh // _L
    mesh = plsc.VectorSubcoreMesh(core_axis_name="c", subcore_axis_name="s",
                                  num_cores=_NC, num_subcores=_NS)

    @functools.partial(
        pl.kernel,
        out_type=jax.ShapeDtypeStruct((NOUT, D), jnp.float32),
        mesh=mesh,
        compiler_params=pltpu.CompilerParams(needs_layout_passes=False),
        scratch_types=[
            pltpu.VMEM((Cw + 8, Dh), jnp.float32),
            pltpu.VMEM((ECH2,), jnp.int32),
            pltpu.VMEM((ECH2,), jnp.int32),
            pltpu.VMEM((ECH2,), jnp.float32),
            pltpu.VMEM((ECH2 + 2 * _L,), jnp.int32),
            pltpu.VMEM((ECH2 + 2 * _L,), jnp.int32),
            pltpu.VMEM((ECH2 + 2 * _L,), jnp.float32),
            pltpu.VMEM((2 * _L,), jnp.int32),
            pltpu.VMEM((2 * _L, Dh), jnp.float32),
        ],
    )
    def agg_kernel(xw, srcp, dstp, exT, agg, acc, srcv, dstv, exv,
                   cs, cd, cex, idxs, rows):
        c = lax.axis_index("c")
        s = lax.axis_index("s")
        if NSLICE == 1:
            h = 0
            w = c * _NS + s
        else:
            h = s % NSLICE
            w = c * (_NS // NSLICE) + s // NSLICE
        iota = lax.iota(jnp.int32, _L)
        tmask = iota >= 0
        zero16 = jnp.zeros((_L,), jnp.float32)

        def pass_body(p, _):
            lo = (p * NWIN + w) * Cw

            def zero_row(r, _z):
                for q in range(nq):
                    acc[r, pl.ds(q * _L, _L)] = zero16
                return 0

            lax.fori_loop(0, Cw + 8, zero_row, 0)

            def chunk_body(ch, _c):
                ebase = pl.multiple_of(ch * ECH2, 128)
                pltpu.sync_copy(srcp.at[pl.ds(ebase, ECH2)], srcv)
                pltpu.sync_copy(dstp.at[pl.ds(ebase, ECH2)], dstv)
                pltpu.sync_copy(exT.at[h, 0, pl.ds(ebase, ECH2)], exv)

                def scan_grp(j, cnt):
                    sv = srcv[pl.ds(j * _L, _L)]
                    dv = dstv[pl.ds(j * _L, _L)]
                    xv = exv[pl.ds(j * _L, _L)]
                    dloc = dv - lo
                    m = (dloc >= 0) & (dloc < Cw)
                    plsc.store_compressed(cs.at[pl.ds(cnt, _L)], sv, mask=m)
                    plsc.store_compressed(cd.at[pl.ds(cnt, _L)], dloc, mask=m)
                    plsc.store_compressed(cex.at[pl.ds(cnt, _L)], xv, mask=m)
                    pc = plsc.all_reduce_population_count(m)
                    return cnt + pc[0]

                cnt = lax.fori_loop(0, ECH2 // _L, scan_grp, 0)
                for pg in range(2):
                    at = pl.ds(cnt + pg * _L, _L)
                    plsc.store_compressed(cs.at[at],
                                          jnp.zeros((_L,), jnp.int32),
                                          mask=tmask)
                    plsc.store_compressed(cd.at[at],
                                          jnp.full((_L,), Cw, jnp.int32),
                                          mask=tmask)
                    plsc.store_compressed(cex.at[at], zero16, mask=tmask)
                nb = (cnt + 2 * _L - 1) // (2 * _L)

                def drain(b, _d):
                    for g in range(2):
                        bi = b * 2 * _L + g * _L + iota
                        idxs[pl.ds(g * _L, _L)] = plsc.load_gather(cs, [bi])
                    if NSLICE == 1:
                        pltpu.sync_copy(xw.at[idxs], rows)
                    else:
                        pltpu.sync_copy(xw.at[idxs, pl.ds(h * Dh, Dh)], rows)
                    for g in range(2):
                        bi = b * 2 * _L + g * _L + iota
                        dlv = plsc.load_gather(cd, [bi])
                        xvv = plsc.load_gather(cex, [bi])

                        def edge_body(e, _e):
                            em = iota == e
                            dl = jnp.max(jnp.where(em, dlv, 0))
                            bc = jnp.full((_L,), jnp.max(jnp.where(em, xvv, 0.0)))
                            for q in range(nq):
                                v = rows[e + g * _L, pl.ds(q * _L, _L)] * bc
                                plsc.addupdate(acc.at[dl, pl.ds(q * _L, _L)], v)
                            return 0

                        lax.fori_loop(0, _L, edge_body, 0)
                    return 0

                lax.fori_loop(0, nb, drain, 0)
                return 0

            lax.fori_loop(0, NCH, chunk_body, 0)
            row0 = pl.multiple_of((p * NWIN + w) * Cw, 8)
            if NSLICE == 1:
                pltpu.sync_copy(acc.at[pl.ds(0, Cw)],
                                agg.at[pl.ds(row0, Cw)])
            else:
                pltpu.sync_copy(acc.at[pl.ds(0, Cw)],
                                agg.at[pl.ds(row0, Cw), pl.ds(h * Dh, Dh)])
            return 0

        lax.fori_loop(0, NPASS, pass_body, 0)

    return agg_kernel


def _edge_phase_jax(a_cat, gvec, src, dst, heads):
    a_src = a_cat[:, :heads]
    a_dst = a_cat[:, 8:8 + heads]
    alpha = a_src[src] + a_dst[dst]
    alpha = jax.nn.leaky_relu(alpha, 0.2)
    ex = jnp.exp(alpha - gvec[None, :heads])
    denom = jax.ops.segment_sum(ex, dst, num_segments=N_NODES)
    return ex, denom


def _agg_jax(ex, xw, src, dst, heads, ch):
    msg = xw.reshape(N_NODES, heads, ch)[src] * ex[:, :, None]
    agg = jax.ops.segment_sum(msg, dst, num_segments=N_NODES)
    return agg.reshape(N_NODES, heads * ch)


def _make_consts():
    e8 = np.repeat(np.eye(HEADS, dtype=np.float32), HID, axis=1)
    k1 = np.tile(np.eye(HEADS, dtype=np.float32), (4, 1))
    ones32 = np.zeros((32, 8), np.float32)
    ones32[:, 0] = 1.0
    e1 = np.zeros((8, HID), np.float32)
    e1[0, :] = 1.0
    return jnp.asarray(e8), jnp.asarray(k1), jnp.asarray(ones32), jnp.asarray(e1)


def kernel(X, edge_index, W1, att_src1, att_dst1, b1, W2, att_src2, att_dst2,
           b2, Wfc, bfc):
    E8, K1, ONES32, E1 = _make_consts()
    src = edge_index[0]
    dst = edge_index[1]

    hh = jnp.arange(HEADS * HID) // HID
    blockdiag = (hh[:, None] == jnp.arange(HEADS)[None, :]).astype(jnp.float32)
    acat = jnp.concatenate(
        [blockdiag * att_src1.reshape(-1)[:, None],
         blockdiag * att_dst1.reshape(-1)[:, None]], axis=1)

    att2cat = jnp.zeros((HID, 16), jnp.float32)
    att2cat = att2cat.at[:, 0].set(att_src2[0]).at[:, 8].set(att_dst2[0])

    srcp = jnp.pad(src, (0, E_PAD - N_EDGES))
    dstp = jnp.pad(dst, (0, E_PAD - N_EDGES))

    xw1, a1cat, gmax1 = _tc1(X, W1, acat)
    g1 = gmax1[0, :HEADS] + gmax1[0, 8:16]
    g1vec = jnp.pad(g1, (0, 8))

    asT1 = jnp.pad(a1cat.T, ((0, 0), (0, N_PAD - N_NODES))).reshape(16, 1, N_PAD)
    exT1, dpart1 = _make_edge_softmax(8)(asT1, srcp, dstp, g1vec)
    agg1 = _make_agg(HEADS, HEADS * HID, 320, 8)(
        xw1, srcp, dstp, exT1)[:N_NODES]
    dp1T = dpart1[:, 0, :N_NODES].T

    xw2, a2cat, gmax2 = _tc2(agg1, dp1T, K1, E8, b1.reshape(1, -1), W2, att2cat)
    g2 = gmax2[0, :1] + gmax2[0, 8:9]
    g2vec = jnp.pad(g2, (0, 15))

    asT2 = jnp.pad(a2cat.T, ((0, 0), (0, N_PAD - N_NODES))).reshape(16, 1, N_PAD)
    exT2, dpart2 = _make_edge_softmax(1)(asT2, srcp, dstp, g2vec)
    agg2 = _make_agg(1, HID, 320, 1)(
        xw2, srcp, dstp, exT2)[:N_NODES]
    dp2T = dpart2[:, 0, :N_NODES].T

    (out,) = _tc3(agg2, dp2T, ONES32, E1, b2.reshape(1, -1), Wfc,
                  bfc.reshape(1, -1))
    return out
